# baseline (speedup 1.0000x reference)
import functools

import jax
import jax.numpy as jnp
from jax import lax
from jax.experimental import pallas as pl
from jax.experimental.pallas import tpu as pltpu
from jax.experimental.pallas import tpu_sc as plsc

N_SAT, N_GW, N_CELL = 5000, 64, 2000
H = 256
HP = 272
NSP = 5008
E_REAL = 85000
NW = 32
NB = 1
NOUT = 168
PERW = NOUT * NB * 16
EP = PERW * NW
ROUNDS = 3

_F32 = jnp.float32


def _mm_relu_body(x_ref, w_ref, b_ref, o_ref):
    acc = jnp.dot(x_ref[...], w_ref[...], preferred_element_type=_F32)
    o_ref[...] = jnp.maximum(acc + b_ref[...], 0.0)


def _encode(x, w, b, blk):
    n, d = x.shape
    h = w.shape[1]
    grid = n // blk
    return pl.pallas_call(
        _mm_relu_body,
        grid=(grid,),
        in_specs=[
            pl.BlockSpec((blk, d), lambda i: (i, 0)),
            pl.BlockSpec((d, h), lambda i: (0, 0)),
            pl.BlockSpec((1, h), lambda i: (0, 0)),
        ],
        out_specs=pl.BlockSpec((blk, h), lambda i: (i, 0)),
        out_shape=jax.ShapeDtypeStruct((n, h), _F32),
    )(x, w, b.reshape(1, h))


def _prep_body(x_ref, w_ref, as_ref, ad_ref, aug_ref, nv_ref):
    hs = jnp.dot(x_ref[...], w_ref[...], preferred_element_type=_F32)
    aug_ref[:, :H] = hs
    lane = lax.broadcasted_iota(jnp.int32, (N_SAT, HP - H), 1)
    aug_ref[:, H:] = jnp.where(lane == 0, 1.0, 0.0)
    es = lax.dot_general(as_ref[...], hs, (((1,), (1,)), ((), ())),
                         preferred_element_type=_F32)
    ed = lax.dot_general(ad_ref[...], hs, (((1,), (1,)), ((), ())),
                         preferred_element_type=_F32)
    t = jnp.max(es) + ed
    mpr = jnp.where(t > 0, t, 0.2 * t)
    nv_ref[0:1, :] = es
    nv_ref[1:2, :] = ed
    nv_ref[2:3, :] = mpr
    nv_ref[3:8, :] = jnp.zeros((5, N_SAT), _F32)


def _prep(x, w, a_s, a_d):
    return pl.pallas_call(
        _prep_body,
        out_shape=(
            jax.ShapeDtypeStruct((N_SAT, HP), _F32),
            jax.ShapeDtypeStruct((8, N_SAT), _F32),
        ),
    )(x, w, a_s.reshape(1, H), a_d.reshape(1, H))


def _merge_gru_body(p_ref, h_ref, b_ref, lg_ref, lb_ref, wih_ref, whh_ref,
                    bih_ref, bhh_ref, o_ref):
    p = p_ref[...]
    num = p[0, :, :H] + p[1, :, :H]
    den = p[0, :, H:H + 1] + p[1, :, H:H + 1]
    out = num / jnp.maximum(den, 1e-16) + b_ref[...]
    mu = jnp.mean(out, axis=-1, keepdims=True)
    var = jnp.mean((out - mu) ** 2, axis=-1, keepdims=True)
    out = (out - mu) / jnp.sqrt(var + 1e-5) * lg_ref[...] + lb_ref[...]
    out = jnp.maximum(out, 0.0)
    h = h_ref[...]
    gi = jnp.dot(out, wih_ref[...], preferred_element_type=_F32) + bih_ref[...]
    gh = jnp.dot(h, whh_ref[...], preferred_element_type=_F32) + bhh_ref[...]
    r = jax.nn.sigmoid(gi[:, :H] + gh[:, :H])
    z = jax.nn.sigmoid(gi[:, H:2 * H] + gh[:, H:2 * H])
    n = jnp.tanh(gi[:, 2 * H:] + r * gh[:, 2 * H:])
    o_ref[...] = (1.0 - z) * n + z * h


def _merge_gru(partials, h_prev, b, lg, lb, wih, whh, bih, bhh):
    blk = 1000
    grid = N_SAT // blk
    return pl.pallas_call(
        _merge_gru_body,
        grid=(grid,),
        in_specs=[
            pl.BlockSpec((2, blk, HP), lambda i: (0, i, 0)),
            pl.BlockSpec((blk, H), lambda i: (i, 0)),
            pl.BlockSpec((1, H), lambda i: (0, 0)),
            pl.BlockSpec((1, H), lambda i: (0, 0)),
            pl.BlockSpec((1, H), lambda i: (0, 0)),
            pl.BlockSpec((H, 3 * H), lambda i: (0, 0)),
            pl.BlockSpec((H, 3 * H), lambda i: (0, 0)),
            pl.BlockSpec((1, 3 * H), lambda i: (0, 0)),
            pl.BlockSpec((1, 3 * H), lambda i: (0, 0)),
        ],
        out_specs=pl.BlockSpec((blk, H), lambda i: (i, 0)),
        out_shape=jax.ShapeDtypeStruct((N_SAT, H), _F32),
    )(partials, h_prev, b.reshape(1, H), lg.reshape(1, H), lb.reshape(1, H),
      wih, whh, bih.reshape(1, 3 * H), bhh.reshape(1, 3 * H))


def _head_softmax_body(h_ref, w_ref, b_ref, v_ref, lo_ref, pr_ref):
    logits = jnp.dot(h_ref[...], w_ref[...], preferred_element_type=_F32)
    logits = logits + b_ref[...]
    logits = jnp.where(v_ref[...] == 0, -1e9, logits)
    lo_ref[...] = logits
    m = jnp.max(logits, axis=-1, keepdims=True)
    e = jnp.exp(logits - m)
    pr_ref[...] = e / jnp.sum(e, axis=-1, keepdims=True)


def _head_sigmoid_body(h_ref, w_ref, b_ref, v_ref, lo_ref, pr_ref):
    logits = jnp.dot(h_ref[...], w_ref[...], preferred_element_type=_F32)
    logits = logits + b_ref[...]
    logits = jnp.where(v_ref[...] == 0, -1e9, logits)
    lo_ref[...] = logits
    pr_ref[...] = jax.nn.sigmoid(logits)


def _head(h, w, b, vis, body, blk):
    n = h.shape[0]
    k = w.shape[1]
    grid = n // blk
    return pl.pallas_call(
        body,
        grid=(grid,),
        in_specs=[
            pl.BlockSpec((blk, H), lambda i: (i, 0)),
            pl.BlockSpec((H, k), lambda i: (0, 0)),
            pl.BlockSpec((1, k), lambda i: (0, 0)),
            pl.BlockSpec((blk, k), lambda i: (i, 0)),
        ],
        out_specs=(
            pl.BlockSpec((blk, k), lambda i: (i, 0)),
            pl.BlockSpec((blk, k), lambda i: (i, 0)),
        ),
        out_shape=(
            jax.ShapeDtypeStruct((n, k), _F32),
            jax.ShapeDtypeStruct((n, k), _F32),
        ),
    )(h, w, b.reshape(1, k), vis)


def _cs_body(x_ref, ht_ref, v_ref, pr_ref):
    scores = jnp.dot(x_ref[...], ht_ref[...], preferred_element_type=_F32)
    scores = jnp.where(v_ref[...] == 0, -1e9, scores)
    m = jnp.max(scores, axis=-1, keepdims=True)
    e = jnp.exp(scores - m)
    pr_ref[...] = e / jnp.sum(e, axis=-1, keepdims=True)


def _head_cs(x_cell, h_t, vis):
    blk = 200
    grid = N_CELL // blk
    return pl.pallas_call(
        _cs_body,
        grid=(grid,),
        in_specs=[
            pl.BlockSpec((blk, H), lambda i: (i, 0)),
            pl.BlockSpec((H, N_SAT), lambda i: (0, 0)),
            pl.BlockSpec((blk, N_SAT), lambda i: (i, 0)),
        ],
        out_specs=pl.BlockSpec((blk, N_SAT), lambda i: (i, 0)),
        out_shape=jax.ShapeDtypeStruct((N_CELL, N_SAT), _F32),
    )(x_cell, h_t, vis)


def _gat_sc_body(aug_hbm, nv_hbm, src_hbm, dst_hbm, out_hbm,
                 src_v, dst_v, es_v, ed_v, mp_v, rows_v,
                 acc, gsem, ssem):
    cid = lax.axis_index("c")
    sid = lax.axis_index("s")
    wid = sid * 2 + cid
    base = wid * PERW
    pltpu.sync_copy(src_hbm.at[pl.ds(base, PERW)], src_v)
    pltpu.sync_copy(dst_hbm.at[pl.ds(base, PERW)], dst_v)
    pltpu.sync_copy(nv_hbm.at[0], es_v)
    pltpu.sync_copy(nv_hbm.at[1], ed_v)
    pltpu.sync_copy(nv_hbm.at[2], mp_v)

    zero = jnp.zeros((16,), _F32)
    for j in range(16):
        for c in range(HP // 16):
            rows_v[0, j, pl.ds(c * 16, 16)] = zero

    nchunks = NSP // 16
    nmine = (nchunks - sid + 15) // 16

    def zbody(i, _):
        k = sid + i * 16
        pltpu.sync_copy(rows_v.at[0], acc.at[pl.ds(k * 16, 16)])
        return 0

    lax.fori_loop(0, nmine, zbody, 0)
    plsc.subcore_barrier()

    def body(g, _):
        s16 = src_v[pl.ds(g * 16, 16)]
        d16 = dst_v[pl.ds(g * 16, 16)]
        pltpu.sync_copy(aug_hbm.at[s16], rows_v.at[0])
        esg = plsc.load_gather(es_v, [s16])
        edg = plsc.load_gather(ed_v, [d16])
        mpg = plsc.load_gather(mp_v, [d16])
        e = esg + edg
        e = jnp.where(e > 0, e, 0.2 * e)
        ex = jnp.exp(e - mpg)
        eid = base + g * 16 + lax.iota(jnp.int32, 16)
        ex = jnp.where(eid < E_REAL, ex, 0.0)
        for j in range(16):
            a = ex[j]
            for c in range(HP // 16):
                sl = pl.ds(c * 16, 16)
                rows_v[0, j, sl] = rows_v[0, j, sl] * a
        pltpu.sync_copy(rows_v.at[0], acc.at[d16], add=True)
        return 0

    lax.fori_loop(0, NOUT, body, 0)
    plsc.subcore_barrier()

    def obody(i, _):
        k = sid + i * 16
        pltpu.sync_copy(acc.at[pl.ds(k * 16, 16)],
                        out_hbm.at[cid, pl.ds(k * 16, 16)])
        return 0

    lax.fori_loop(0, nmine, obody, 0)


def _gat_edge(aug, nv, src, dst):
    mesh = plsc.VectorSubcoreMesh(core_axis_name="c", subcore_axis_name="s")
    f = functools.partial(
        pl.kernel,
        mesh=mesh,
        compiler_params=pltpu.CompilerParams(
            needs_layout_passes=False,
            use_tc_tiling_on_sc=False,
        ),
        out_type=jax.ShapeDtypeStruct((2, NSP, HP), _F32),
        scratch_types=[
            pltpu.VMEM((PERW,), jnp.int32),
            pltpu.VMEM((PERW,), jnp.int32),
            pltpu.VMEM((N_SAT,), _F32),
            pltpu.VMEM((N_SAT,), _F32),
            pltpu.VMEM((N_SAT,), _F32),
            pltpu.VMEM((NB, 16, HP), _F32),
            pltpu.VMEM_SHARED((NSP, HP), _F32),
            pltpu.SemaphoreType.DMA,
            pltpu.SemaphoreType.DMA,
        ],
    )(_gat_sc_body)
    return f(aug, nv, src, dst)


def kernel(sat_x, gateway_x, cell_x, ei_ss, sg_src, sg_dst, gc_src, gc_dst,
           sc_src, sc_dst, vis_sat_gateway, vis_sat_cell, vis_cell_sat,
           params):
    loop = jnp.arange(N_SAT, dtype=jnp.int32)
    padz = jnp.zeros((EP - E_REAL,), jnp.int32)
    src = jnp.concatenate([ei_ss[0].astype(jnp.int32), loop, padz])
    dst = jnp.concatenate([ei_ss[1].astype(jnp.int32), loop, padz])

    x_sat = _encode(sat_x, params['enc_sat_W'], params['enc_sat_b'], 1000)
    x_cell = _encode(cell_x, params['enc_cell_W'], params['enc_cell_b'], 1000)

    gp = params['gat_ss']
    h = x_sat
    x_cur = x_sat
    for _ in range(ROUNDS):
        aug, nv = _prep(x_cur, gp['W'], gp['a_s'], gp['a_d'])
        partials = _gat_edge(aug, nv, src, dst)
        h = _merge_gru(partials, h, gp['b'], params['ln_g'], params['ln_b'],
                       params['gru_Wih'], params['gru_Whh'],
                       params['gru_bih'], params['gru_bhh'])
        x_cur = h

    sg_logits, sg_probs = _head(h, params['head_sg_W'], params['head_sg_b'],
                                vis_sat_gateway, _head_softmax_body, 1000)
    sc_logits, sc_probs = _head(h, params['head_sc_W'], params['head_sc_b'],
                                vis_sat_cell, _head_sigmoid_body, 200)
    cs_probs = _head_cs(x_cell, h.T, vis_cell_sat)
    return (sg_logits, sc_logits, sg_probs, sc_probs, cs_probs, h)


# speedup vs baseline: 1.2473x; 1.2473x over previous
import functools

import jax
import jax.numpy as jnp
from jax import lax
from jax.experimental import pallas as pl
from jax.experimental.pallas import tpu as pltpu
from jax.experimental.pallas import tpu_sc as plsc

N_SAT, N_GW, N_CELL = 5000, 64, 2000
H = 256
HP = 272
NSP = 5008
E_REAL = 85000
NW = 32
NB = 4
NOUT = 42
PERW = NOUT * NB * 16
EP = PERW * NW
ROUNDS = 3

_F32 = jnp.float32


def _mm_relu_body(x_ref, w_ref, b_ref, o_ref):
    acc = jnp.dot(x_ref[...], w_ref[...], preferred_element_type=_F32)
    o_ref[...] = jnp.maximum(acc + b_ref[...], 0.0)


def _encode(x, w, b, blk):
    n, d = x.shape
    h = w.shape[1]
    grid = n // blk
    return pl.pallas_call(
        _mm_relu_body,
        grid=(grid,),
        in_specs=[
            pl.BlockSpec((blk, d), lambda i: (i, 0)),
            pl.BlockSpec((d, h), lambda i: (0, 0)),
            pl.BlockSpec((1, h), lambda i: (0, 0)),
        ],
        out_specs=pl.BlockSpec((blk, h), lambda i: (i, 0)),
        out_shape=jax.ShapeDtypeStruct((n, h), _F32),
    )(x, w, b.reshape(1, h))


def _prep_body(x_ref, w_ref, as_ref, ad_ref, aug_ref, nv_ref):
    hs = jnp.dot(x_ref[...], w_ref[...], preferred_element_type=_F32)
    aug_ref[:N_SAT, :H] = hs
    aug_ref[N_SAT:, :H] = jnp.zeros((NSP - N_SAT, H), _F32)
    lane = lax.broadcasted_iota(jnp.int32, (NSP, HP - H), 1)
    aug_ref[:, H:] = jnp.where(lane == 0, 1.0, 0.0)
    es = lax.dot_general(as_ref[...], hs, (((1,), (1,)), ((), ())),
                         preferred_element_type=_F32)
    ed = lax.dot_general(ad_ref[...], hs, (((1,), (1,)), ((), ())),
                         preferred_element_type=_F32)
    t = jnp.max(es) + ed
    mpr = jnp.where(t > 0, t, 0.2 * t)
    nv_ref[0:1, :] = es
    nv_ref[1:2, :] = ed
    nv_ref[2:3, :] = mpr
    nv_ref[3:8, :] = jnp.zeros((5, N_SAT), _F32)


def _prep(x, w, a_s, a_d):
    return pl.pallas_call(
        _prep_body,
        out_shape=(
            jax.ShapeDtypeStruct((NSP, HP), _F32),
            jax.ShapeDtypeStruct((8, N_SAT), _F32),
        ),
    )(x, w, a_s.reshape(1, H), a_d.reshape(1, H))


def _merge_gru_body(p_ref, h_ref, b_ref, lg_ref, lb_ref, wih_ref, whh_ref,
                    bih_ref, bhh_ref, o_ref):
    p = p_ref[...]
    num = p[0, :, :H] + p[1, :, :H]
    den = p[0, :, H:H + 1] + p[1, :, H:H + 1]
    out = num / jnp.maximum(den, 1e-16) + b_ref[...]
    mu = jnp.mean(out, axis=-1, keepdims=True)
    var = jnp.mean((out - mu) ** 2, axis=-1, keepdims=True)
    out = (out - mu) / jnp.sqrt(var + 1e-5) * lg_ref[...] + lb_ref[...]
    out = jnp.maximum(out, 0.0)
    h = h_ref[...]
    gi = jnp.dot(out, wih_ref[...], preferred_element_type=_F32) + bih_ref[...]
    gh = jnp.dot(h, whh_ref[...], preferred_element_type=_F32) + bhh_ref[...]
    r = jax.nn.sigmoid(gi[:, :H] + gh[:, :H])
    z = jax.nn.sigmoid(gi[:, H:2 * H] + gh[:, H:2 * H])
    n = jnp.tanh(gi[:, 2 * H:] + r * gh[:, 2 * H:])
    o_ref[...] = (1.0 - z) * n + z * h


def _merge_gru(partials, h_prev, b, lg, lb, wih, whh, bih, bhh):
    blk = 1000
    grid = N_SAT // blk
    return pl.pallas_call(
        _merge_gru_body,
        grid=(grid,),
        in_specs=[
            pl.BlockSpec((2, blk, HP), lambda i: (0, i, 0)),
            pl.BlockSpec((blk, H), lambda i: (i, 0)),
            pl.BlockSpec((1, H), lambda i: (0, 0)),
            pl.BlockSpec((1, H), lambda i: (0, 0)),
            pl.BlockSpec((1, H), lambda i: (0, 0)),
            pl.BlockSpec((H, 3 * H), lambda i: (0, 0)),
            pl.BlockSpec((H, 3 * H), lambda i: (0, 0)),
            pl.BlockSpec((1, 3 * H), lambda i: (0, 0)),
            pl.BlockSpec((1, 3 * H), lambda i: (0, 0)),
        ],
        out_specs=pl.BlockSpec((blk, H), lambda i: (i, 0)),
        out_shape=jax.ShapeDtypeStruct((N_SAT, H), _F32),
    )(partials, h_prev, b.reshape(1, H), lg.reshape(1, H), lb.reshape(1, H),
      wih, whh, bih.reshape(1, 3 * H), bhh.reshape(1, 3 * H))


def _head_softmax_body(h_ref, w_ref, b_ref, v_ref, lo_ref, pr_ref):
    logits = jnp.dot(h_ref[...], w_ref[...], preferred_element_type=_F32)
    logits = logits + b_ref[...]
    logits = jnp.where(v_ref[...] == 0, -1e9, logits)
    lo_ref[...] = logits
    m = jnp.max(logits, axis=-1, keepdims=True)
    e = jnp.exp(logits - m)
    pr_ref[...] = e / jnp.sum(e, axis=-1, keepdims=True)


def _head_sigmoid_body(h_ref, w_ref, b_ref, v_ref, lo_ref, pr_ref):
    logits = jnp.dot(h_ref[...], w_ref[...], preferred_element_type=_F32)
    logits = logits + b_ref[...]
    logits = jnp.where(v_ref[...] == 0, -1e9, logits)
    lo_ref[...] = logits
    pr_ref[...] = jax.nn.sigmoid(logits)


def _head(h, w, b, vis, body, blk):
    n = h.shape[0]
    k = w.shape[1]
    grid = n // blk
    return pl.pallas_call(
        body,
        grid=(grid,),
        in_specs=[
            pl.BlockSpec((blk, H), lambda i: (i, 0)),
            pl.BlockSpec((H, k), lambda i: (0, 0)),
            pl.BlockSpec((1, k), lambda i: (0, 0)),
            pl.BlockSpec((blk, k), lambda i: (i, 0)),
        ],
        out_specs=(
            pl.BlockSpec((blk, k), lambda i: (i, 0)),
            pl.BlockSpec((blk, k), lambda i: (i, 0)),
        ),
        out_shape=(
            jax.ShapeDtypeStruct((n, k), _F32),
            jax.ShapeDtypeStruct((n, k), _F32),
        ),
    )(h, w, b.reshape(1, k), vis)


def _cs_body(x_ref, ht_ref, v_ref, pr_ref):
    scores = jnp.dot(x_ref[...], ht_ref[...], preferred_element_type=_F32)
    scores = jnp.where(v_ref[...] == 0, -1e9, scores)
    m = jnp.max(scores, axis=-1, keepdims=True)
    e = jnp.exp(scores - m)
    pr_ref[...] = e / jnp.sum(e, axis=-1, keepdims=True)


def _head_cs(x_cell, h_t, vis):
    blk = 200
    grid = N_CELL // blk
    return pl.pallas_call(
        _cs_body,
        grid=(grid,),
        in_specs=[
            pl.BlockSpec((blk, H), lambda i: (i, 0)),
            pl.BlockSpec((H, N_SAT), lambda i: (0, 0)),
            pl.BlockSpec((blk, N_SAT), lambda i: (i, 0)),
        ],
        out_specs=pl.BlockSpec((blk, N_SAT), lambda i: (i, 0)),
        out_shape=jax.ShapeDtypeStruct((N_CELL, N_SAT), _F32),
    )(x_cell, h_t, vis)


def _gat_sc_body(aug_hbm, nv_hbm, src_hbm, dst_hbm, out_hbm,
                 src_v, dst_v, es_v, ed_v, mp_v, rows_v,
                 acc, sm0, sm1, sm2, sm3):
    cid = lax.axis_index("c")
    sid = lax.axis_index("s")
    wid = sid * 2 + cid
    base = wid * PERW
    pltpu.sync_copy(src_hbm.at[pl.ds(base, PERW)], src_v)
    pltpu.sync_copy(dst_hbm.at[pl.ds(base, PERW)], dst_v)
    pltpu.sync_copy(nv_hbm.at[0], es_v)
    pltpu.sync_copy(nv_hbm.at[1], ed_v)
    pltpu.sync_copy(nv_hbm.at[2], mp_v)

    zero = jnp.zeros((16,), _F32)
    for j in range(16):
        for c in range(HP // 16):
            rows_v[0, j, pl.ds(c * 16, 16)] = zero

    nchunks = NSP // 16
    nmine = (nchunks - sid + 15) // 16

    def zbody(i, _):
        k = sid + i * 16
        pltpu.sync_copy(rows_v.at[0], acc.at[pl.ds(k * 16, 16)])
        return 0

    lax.fori_loop(0, nmine, zbody, 0)
    plsc.subcore_barrier()

    sems = [sm0, sm1, sm2, sm3]

    def body(g0, _):
        exs, d16s, gds = [], [], []
        for b in range(NB):
            g = g0 * NB + b
            s16 = src_v[pl.ds(g * 16, 16)]
            d16 = dst_v[pl.ds(g * 16, 16)]
            gds.append(pltpu.async_copy(
                aug_hbm.at[src_v.at[pl.ds(g * 16, 16)]], rows_v.at[b],
                sems[b]))
            esg = plsc.load_gather(es_v, [s16])
            edg = plsc.load_gather(ed_v, [d16])
            mpg = plsc.load_gather(mp_v, [d16])
            e = esg + edg
            e = jnp.where(e > 0, e, 0.2 * e)
            ex = jnp.exp(e - mpg)
            eid = base + g * 16 + lax.iota(jnp.int32, 16)
            exs.append(jnp.where(eid < E_REAL, ex, 0.0))
            d16s.append(d16)
        for b in range(NB):
            gds[b].wait()
            ex = exs[b]
            for j in range(16):
                a = ex[j]
                for c in range(HP // 16):
                    sl = pl.ds(c * 16, 16)
                    rows_v[b, j, sl] = rows_v[b, j, sl] * a
            pltpu.sync_copy(rows_v.at[b], acc.at[d16s[b]], add=True)
        return 0

    lax.fori_loop(0, NOUT, body, 0)
    plsc.subcore_barrier()

    def obody(i, _):
        k = sid + i * 16
        pltpu.sync_copy(acc.at[pl.ds(k * 16, 16)],
                        out_hbm.at[cid, pl.ds(k * 16, 16)])
        return 0

    lax.fori_loop(0, nmine, obody, 0)


def _gat_edge(aug, nv, src, dst):
    mesh = plsc.VectorSubcoreMesh(core_axis_name="c", subcore_axis_name="s")
    f = functools.partial(
        pl.kernel,
        mesh=mesh,
        compiler_params=pltpu.CompilerParams(
            needs_layout_passes=False,
            use_tc_tiling_on_sc=False,
        ),
        out_type=jax.ShapeDtypeStruct((2, NSP, HP), _F32),
        scratch_types=[
            pltpu.VMEM((PERW,), jnp.int32),
            pltpu.VMEM((PERW,), jnp.int32),
            pltpu.VMEM((N_SAT,), _F32),
            pltpu.VMEM((N_SAT,), _F32),
            pltpu.VMEM((N_SAT,), _F32),
            pltpu.VMEM((NB, 16, HP), _F32),
            pltpu.VMEM_SHARED((NSP, HP), _F32),
            pltpu.SemaphoreType.DMA,
            pltpu.SemaphoreType.DMA,
            pltpu.SemaphoreType.DMA,
            pltpu.SemaphoreType.DMA,
        ],
    )(_gat_sc_body)
    return f(aug, nv, src, dst)


def kernel(sat_x, gateway_x, cell_x, ei_ss, sg_src, sg_dst, gc_src, gc_dst,
           sc_src, sc_dst, vis_sat_gateway, vis_sat_cell, vis_cell_sat,
           params):
    loop = jnp.arange(N_SAT, dtype=jnp.int32)
    padz = jnp.zeros((EP - E_REAL,), jnp.int32)
    src = jnp.concatenate([ei_ss[0].astype(jnp.int32), loop, padz])
    dst = jnp.concatenate([ei_ss[1].astype(jnp.int32), loop, padz])

    x_sat = _encode(sat_x, params['enc_sat_W'], params['enc_sat_b'], 1000)
    x_cell = _encode(cell_x, params['enc_cell_W'], params['enc_cell_b'], 1000)

    gp = params['gat_ss']
    h = x_sat
    x_cur = x_sat
    for _ in range(ROUNDS):
        aug, nv = _prep(x_cur, gp['W'], gp['a_s'], gp['a_d'])
        partials = _gat_edge(aug, nv, src, dst)
        h = _merge_gru(partials, h, gp['b'], params['ln_g'], params['ln_b'],
                       params['gru_Wih'], params['gru_Whh'],
                       params['gru_bih'], params['gru_bhh'])
        x_cur = h

    sg_logits, sg_probs = _head(h, params['head_sg_W'], params['head_sg_b'],
                                vis_sat_gateway, _head_softmax_body, 1000)
    sc_logits, sc_probs = _head(h, params['head_sc_W'], params['head_sc_b'],
                                vis_sat_cell, _head_sigmoid_body, 200)
    cs_probs = _head_cs(x_cell, h.T, vis_cell_sat)
    return (sg_logits, sc_logits, sg_probs, sc_probs, cs_probs, h)


# speedup vs baseline: 1.3010x; 1.0431x over previous
import functools

import jax
import jax.numpy as jnp
from jax import lax
from jax.experimental import pallas as pl
from jax.experimental.pallas import tpu as pltpu
from jax.experimental.pallas import tpu_sc as plsc

N_SAT, N_GW, N_CELL = 5000, 64, 2000
H = 256
HP = 272
NSP = 5008
E_REAL = 85000
NW = 32
NB = 6
NOUT = 28
PERW = NOUT * NB * 16
EP = PERW * NW
ROUNDS = 3

_F32 = jnp.float32


def _mm_relu_body(x_ref, w_ref, b_ref, o_ref):
    acc = jnp.dot(x_ref[...], w_ref[...], preferred_element_type=_F32)
    o_ref[...] = jnp.maximum(acc + b_ref[...], 0.0)


def _encode(x, w, b, blk):
    n, d = x.shape
    h = w.shape[1]
    grid = n // blk
    return pl.pallas_call(
        _mm_relu_body,
        grid=(grid,),
        in_specs=[
            pl.BlockSpec((blk, d), lambda i: (i, 0)),
            pl.BlockSpec((d, h), lambda i: (0, 0)),
            pl.BlockSpec((1, h), lambda i: (0, 0)),
        ],
        out_specs=pl.BlockSpec((blk, h), lambda i: (i, 0)),
        out_shape=jax.ShapeDtypeStruct((n, h), _F32),
    )(x, w, b.reshape(1, h))


def _prep_body(x_ref, w_ref, as_ref, ad_ref, aug_ref, nv_ref):
    hs = jnp.dot(x_ref[...], w_ref[...], preferred_element_type=_F32)
    aug_ref[:N_SAT, :H] = hs
    aug_ref[N_SAT:, :H] = jnp.zeros((NSP - N_SAT, H), _F32)
    lane = lax.broadcasted_iota(jnp.int32, (NSP, HP - H), 1)
    aug_ref[:, H:] = jnp.where(lane == 0, 1.0, 0.0)
    es = lax.dot_general(as_ref[...], hs, (((1,), (1,)), ((), ())),
                         preferred_element_type=_F32)
    ed = lax.dot_general(ad_ref[...], hs, (((1,), (1,)), ((), ())),
                         preferred_element_type=_F32)
    mxe = jnp.max(es)
    nv_ref[0:1, :] = es
    nv_ref[1:2, :] = ed
    nv_ref[2:3, :] = jnp.zeros((1, N_SAT), _F32) + mxe
    nv_ref[3:8, :] = jnp.zeros((5, N_SAT), _F32)


def _prep(x, w, a_s, a_d):
    return pl.pallas_call(
        _prep_body,
        out_shape=(
            jax.ShapeDtypeStruct((NSP, HP), _F32),
            jax.ShapeDtypeStruct((8, N_SAT), _F32),
        ),
    )(x, w, a_s.reshape(1, H), a_d.reshape(1, H))


def _merge_gru_body(p_ref, h_ref, b_ref, lg_ref, lb_ref, wih_ref, whh_ref,
                    bih_ref, bhh_ref, o_ref):
    p = p_ref[...]
    num = p[0, :, :H] + p[1, :, :H]
    den = p[0, :, H:H + 1] + p[1, :, H:H + 1]
    out = num / jnp.maximum(den, 1e-16) + b_ref[...]
    mu = jnp.mean(out, axis=-1, keepdims=True)
    var = jnp.mean((out - mu) ** 2, axis=-1, keepdims=True)
    out = (out - mu) / jnp.sqrt(var + 1e-5) * lg_ref[...] + lb_ref[...]
    out = jnp.maximum(out, 0.0)
    h = h_ref[...]
    gi = jnp.dot(out, wih_ref[...], preferred_element_type=_F32) + bih_ref[...]
    gh = jnp.dot(h, whh_ref[...], preferred_element_type=_F32) + bhh_ref[...]
    r = jax.nn.sigmoid(gi[:, :H] + gh[:, :H])
    z = jax.nn.sigmoid(gi[:, H:2 * H] + gh[:, H:2 * H])
    n = jnp.tanh(gi[:, 2 * H:] + r * gh[:, 2 * H:])
    o_ref[...] = (1.0 - z) * n + z * h


def _merge_gru(partials, h_prev, b, lg, lb, wih, whh, bih, bhh):
    blk = 1000
    grid = N_SAT // blk
    return pl.pallas_call(
        _merge_gru_body,
        grid=(grid,),
        in_specs=[
            pl.BlockSpec((2, blk, HP), lambda i: (0, i, 0)),
            pl.BlockSpec((blk, H), lambda i: (i, 0)),
            pl.BlockSpec((1, H), lambda i: (0, 0)),
            pl.BlockSpec((1, H), lambda i: (0, 0)),
            pl.BlockSpec((1, H), lambda i: (0, 0)),
            pl.BlockSpec((H, 3 * H), lambda i: (0, 0)),
            pl.BlockSpec((H, 3 * H), lambda i: (0, 0)),
            pl.BlockSpec((1, 3 * H), lambda i: (0, 0)),
            pl.BlockSpec((1, 3 * H), lambda i: (0, 0)),
        ],
        out_specs=pl.BlockSpec((blk, H), lambda i: (i, 0)),
        out_shape=jax.ShapeDtypeStruct((N_SAT, H), _F32),
    )(partials, h_prev, b.reshape(1, H), lg.reshape(1, H), lb.reshape(1, H),
      wih, whh, bih.reshape(1, 3 * H), bhh.reshape(1, 3 * H))


def _head_softmax_body(h_ref, w_ref, b_ref, v_ref, lo_ref, pr_ref):
    logits = jnp.dot(h_ref[...], w_ref[...], preferred_element_type=_F32)
    logits = logits + b_ref[...]
    logits = jnp.where(v_ref[...] == 0, -1e9, logits)
    lo_ref[...] = logits
    m = jnp.max(logits, axis=-1, keepdims=True)
    e = jnp.exp(logits - m)
    pr_ref[...] = e / jnp.sum(e, axis=-1, keepdims=True)


def _head_sigmoid_body(h_ref, w_ref, b_ref, v_ref, lo_ref, pr_ref):
    logits = jnp.dot(h_ref[...], w_ref[...], preferred_element_type=_F32)
    logits = logits + b_ref[...]
    logits = jnp.where(v_ref[...] == 0, -1e9, logits)
    lo_ref[...] = logits
    pr_ref[...] = jax.nn.sigmoid(logits)


def _head(h, w, b, vis, body, blk):
    n = h.shape[0]
    k = w.shape[1]
    grid = n // blk
    return pl.pallas_call(
        body,
        grid=(grid,),
        in_specs=[
            pl.BlockSpec((blk, H), lambda i: (i, 0)),
            pl.BlockSpec((H, k), lambda i: (0, 0)),
            pl.BlockSpec((1, k), lambda i: (0, 0)),
            pl.BlockSpec((blk, k), lambda i: (i, 0)),
        ],
        out_specs=(
            pl.BlockSpec((blk, k), lambda i: (i, 0)),
            pl.BlockSpec((blk, k), lambda i: (i, 0)),
        ),
        out_shape=(
            jax.ShapeDtypeStruct((n, k), _F32),
            jax.ShapeDtypeStruct((n, k), _F32),
        ),
    )(h, w, b.reshape(1, k), vis)


def _cs_body(x_ref, ht_ref, v_ref, pr_ref):
    scores = jnp.dot(x_ref[...], ht_ref[...], preferred_element_type=_F32)
    scores = jnp.where(v_ref[...] == 0, -1e9, scores)
    m = jnp.max(scores, axis=-1, keepdims=True)
    e = jnp.exp(scores - m)
    pr_ref[...] = e / jnp.sum(e, axis=-1, keepdims=True)


def _head_cs(x_cell, h_t, vis):
    blk = 200
    grid = N_CELL // blk
    return pl.pallas_call(
        _cs_body,
        grid=(grid,),
        in_specs=[
            pl.BlockSpec((blk, H), lambda i: (i, 0)),
            pl.BlockSpec((H, N_SAT), lambda i: (0, 0)),
            pl.BlockSpec((blk, N_SAT), lambda i: (i, 0)),
        ],
        out_specs=pl.BlockSpec((blk, N_SAT), lambda i: (i, 0)),
        out_shape=jax.ShapeDtypeStruct((N_CELL, N_SAT), _F32),
    )(x_cell, h_t, vis)


def _gat_sc_body(aug_hbm, nv_hbm, src_hbm, dst_hbm, out_hbm,
                 src_v, dst_v, es_v, ed_v, mx_v, rows_v,
                 acc, sm0, sm1, sm2, sm3, sm4, sm5):
    cid = lax.axis_index("c")
    sid = lax.axis_index("s")
    wid = sid * 2 + cid
    base = wid * PERW
    pltpu.sync_copy(src_hbm.at[pl.ds(base, PERW)], src_v)
    pltpu.sync_copy(dst_hbm.at[pl.ds(base, PERW)], dst_v)
    pltpu.sync_copy(nv_hbm.at[0], es_v)
    pltpu.sync_copy(nv_hbm.at[1], ed_v)
    pltpu.sync_copy(nv_hbm.at[2, pl.ds(0, 16)], mx_v)

    zero = jnp.zeros((16,), _F32)
    for j in range(16):
        for c in range(HP // 16):
            rows_v[0, j, pl.ds(c * 16, 16)] = zero

    nchunks = NSP // 16
    nmine = (nchunks - sid + 15) // 16

    def zbody(i, _):
        k = sid + i * 16
        pltpu.sync_copy(rows_v.at[0], acc.at[pl.ds(k * 16, 16)])
        return 0

    lax.fori_loop(0, nmine, zbody, 0)
    plsc.subcore_barrier()

    sems = [sm0, sm1, sm2, sm3, sm4, sm5]
    mx16 = mx_v[...]

    def body(g0, _):
        exs, d16s, gds = [], [], []
        for b in range(NB):
            g = g0 * NB + b
            s16 = src_v[pl.ds(g * 16, 16)]
            d16 = dst_v[pl.ds(g * 16, 16)]
            gds.append(pltpu.async_copy(
                aug_hbm.at[src_v.at[pl.ds(g * 16, 16)]], rows_v.at[b],
                sems[b]))
            esg = plsc.load_gather(es_v, [s16])
            edg = plsc.load_gather(ed_v, [d16])
            t = mx16 + edg
            mpg = jnp.where(t > 0, t, 0.2 * t)
            e = esg + edg
            e = jnp.where(e > 0, e, 0.2 * e)
            ex = jnp.exp(e - mpg)
            eid = base + g * 16 + lax.iota(jnp.int32, 16)
            exs.append(jnp.where(eid < E_REAL, ex, 0.0))
            d16s.append(d16)
        for b in range(NB):
            gds[b].wait()
            ex = exs[b]
            for j in range(16):
                a = ex[j]
                for c in range(HP // 16):
                    sl = pl.ds(c * 16, 16)
                    rows_v[b, j, sl] = rows_v[b, j, sl] * a
            pltpu.sync_copy(rows_v.at[b], acc.at[d16s[b]], add=True)
        return 0

    lax.fori_loop(0, NOUT, body, 0)
    plsc.subcore_barrier()

    def obody(i, _):
        k = sid + i * 16
        pltpu.sync_copy(acc.at[pl.ds(k * 16, 16)],
                        out_hbm.at[cid, pl.ds(k * 16, 16)])
        return 0

    lax.fori_loop(0, nmine, obody, 0)


def _gat_edge(aug, nv, src, dst):
    mesh = plsc.VectorSubcoreMesh(core_axis_name="c", subcore_axis_name="s")
    f = functools.partial(
        pl.kernel,
        mesh=mesh,
        compiler_params=pltpu.CompilerParams(
            needs_layout_passes=False,
            use_tc_tiling_on_sc=False,
        ),
        out_type=jax.ShapeDtypeStruct((2, NSP, HP), _F32),
        scratch_types=[
            pltpu.VMEM((PERW,), jnp.int32),
            pltpu.VMEM((PERW,), jnp.int32),
            pltpu.VMEM((N_SAT,), _F32),
            pltpu.VMEM((N_SAT,), _F32),
            pltpu.VMEM((16,), _F32),
            pltpu.VMEM((NB, 16, HP), _F32),
            pltpu.VMEM_SHARED((NSP, HP), _F32),
            pltpu.SemaphoreType.DMA,
            pltpu.SemaphoreType.DMA,
            pltpu.SemaphoreType.DMA,
            pltpu.SemaphoreType.DMA,
            pltpu.SemaphoreType.DMA,
            pltpu.SemaphoreType.DMA,
        ],
    )(_gat_sc_body)
    return f(aug, nv, src, dst)


def kernel(sat_x, gateway_x, cell_x, ei_ss, sg_src, sg_dst, gc_src, gc_dst,
           sc_src, sc_dst, vis_sat_gateway, vis_sat_cell, vis_cell_sat,
           params):
    loop = jnp.arange(N_SAT, dtype=jnp.int32)
    padz = jnp.zeros((EP - E_REAL,), jnp.int32)
    src = jnp.concatenate([ei_ss[0].astype(jnp.int32), loop, padz])
    dst = jnp.concatenate([ei_ss[1].astype(jnp.int32), loop, padz])

    x_sat = _encode(sat_x, params['enc_sat_W'], params['enc_sat_b'], 1000)
    x_cell = _encode(cell_x, params['enc_cell_W'], params['enc_cell_b'], 1000)

    gp = params['gat_ss']
    h = x_sat
    x_cur = x_sat
    for _ in range(ROUNDS):
        aug, nv = _prep(x_cur, gp['W'], gp['a_s'], gp['a_d'])
        partials = _gat_edge(aug, nv, src, dst)
        h = _merge_gru(partials, h, gp['b'], params['ln_g'], params['ln_b'],
                       params['gru_Wih'], params['gru_Whh'],
                       params['gru_bih'], params['gru_bhh'])
        x_cur = h

    sg_logits, sg_probs = _head(h, params['head_sg_W'], params['head_sg_b'],
                                vis_sat_gateway, _head_softmax_body, 1000)
    sc_logits, sc_probs = _head(h, params['head_sc_W'], params['head_sc_b'],
                                vis_sat_cell, _head_sigmoid_body, 200)
    cs_probs = _head_cs(x_cell, h.T, vis_cell_sat)
    return (sg_logits, sc_logits, sg_probs, sc_probs, cs_probs, h)
